# baseline (speedup 1.0000x reference)
import functools

import jax
import jax.numpy as jnp
from jax import lax
from jax.experimental import pallas as pl
from jax.experimental.pallas import tpu as pltpu
from jax.experimental.pallas import tpu_sc as plsc

_N = 10000
_K = 32
_D = 128
_KPT = 512
_NPAD = 10240
_NW = 32
_PTS_W = _NPAD // _NW
_CPTS = 4
_NCH = _PTS_W // _CPTS
_NBUF = 4
_NEG = -3.0e38
_HI = jax.lax.Precision.HIGHEST


def _sc_neighbor_stats(features, nbflat):
    mesh = plsc.VectorSubcoreMesh(core_axis_name="c", subcore_axis_name="s")
    ptsw = _NPAD // 16
    nch = ptsw // _CPTS
    crows = _CPTS * _K

    @functools.partial(
        pl.kernel,
        mesh=mesh,
        compiler_params=pltpu.CompilerParams(needs_layout_passes=False),
        out_type=[
            jax.ShapeDtypeStruct((_NPAD, _D), jnp.float32),
            jax.ShapeDtypeStruct((_NPAD, _D), jnp.float32),
        ],
        scratch_types=[
            pltpu.VMEM((ptsw * _K,), jnp.int32),
            pltpu.VMEM((_NBUF, crows, _D), jnp.float32),
            pltpu.VMEM((_CPTS, _D), jnp.float32),
            pltpu.VMEM((_CPTS, _D), jnp.float32),
            pltpu.SemaphoreType.DMA,
            pltpu.SemaphoreType.DMA,
            pltpu.SemaphoreType.DMA,
            pltpu.SemaphoreType.DMA,
        ],
    )
    def k(feat_hbm, nb_hbm, mean_hbm, max_hbm,
          nb_v, rows_v, mean_v, max_v, s0, s1, s2, s3):
        sems = (s0, s1, s2, s3)
        cid = lax.axis_index("c")
        sid = lax.axis_index("s")

        @pl.when(cid == 0)
        def _work():
            start = sid * ptsw
            pltpu.sync_copy(nb_hbm.at[pl.ds(start * _K, ptsw * _K)], nb_v)

            def fire(c, b):
                idx_sl = nb_v.at[pl.ds(c * crows, crows)]
                pltpu.async_copy(feat_hbm.at[idx_sl], rows_v.at[b], sems[b])

            def wait(c, b):
                idx_sl = nb_v.at[pl.ds(c * crows, crows)]
                pltpu.make_async_copy(feat_hbm.at[idx_sl], rows_v.at[b],
                                      sems[b]).wait()

            def compute_chunk(c, b):
                base = start + c * _CPTS

                def pt_body(p, carry):
                    accs = [None] * 8
                    accm = [None] * 8
                    crs = []
                    for r in range(_K):
                        vs = [rows_v[b, p * _K + r, pl.ds(j * 16, 16)]
                              for j in range(8)]
                        rs01 = vs[0] + vs[1]
                        rs23 = vs[2] + vs[3]
                        rs45 = vs[4] + vs[5]
                        rs67 = vs[6] + vs[7]
                        rowsum = jnp.sum((rs01 + rs23) + (rs45 + rs67))
                        crs.append(jnp.where(rowsum != 0.0,
                                             jnp.float32(1.0),
                                             jnp.float32(0.0)))
                        if r == 0:
                            accs = list(vs)
                            accm = list(vs)
                        else:
                            accs = [a + v for a, v in zip(accs, vs)]
                            accm = [jnp.maximum(a, v)
                                    for a, v in zip(accm, vs)]
                    while len(crs) > 1:
                        crs = [crs[i] + crs[i + 1]
                               for i in range(0, len(crs), 2)]
                    cs = jnp.maximum(crs[0], jnp.float32(1.0))
                    for j in range(8):
                        mean_v[p, pl.ds(j * 16, 16)] = accs[j] / cs
                        max_v[p, pl.ds(j * 16, 16)] = accm[j]
                    return carry

                lax.fori_loop(0, _CPTS, pt_body, jnp.int32(0))
                pltpu.sync_copy(mean_v, mean_hbm.at[pl.ds(base, _CPTS)])
                pltpu.sync_copy(max_v, max_hbm.at[pl.ds(base, _CPTS)])

            for b in range(_NBUF):
                fire(jnp.int32(b), b)

            def group_body(g, carry):
                for b in range(_NBUF):
                    c = g * _NBUF + b
                    wait(c, b)
                    compute_chunk(c, b)
                    fire(lax.rem(c + _NBUF, nch), b)
                return carry

            lax.fori_loop(0, nch // _NBUF, group_body, jnp.int32(0))
            for b in range(_NBUF):
                wait(jnp.int32(b), b)

    return k(features, nbflat)


def _score_body(x_ref, mean_ref, mx_ref, out_ref):
    x = x_ref[...]
    mean = mean_ref[...]
    mx = mx_ref[...]
    rowmax = jnp.max(x, axis=1, keepdims=True)
    beta = x / (rowmax + 1e-6)
    alpha = jax.nn.softplus(x - mean)
    score_map = jnp.max(alpha * beta, axis=1)
    detected = jnp.max((x == mx).astype(jnp.float32), axis=1)
    out_ref[...] = (score_map * detected)[None, None, :]


def _tc_score(feat_pad, mean_feat, max_feat):
    return pl.pallas_call(
        _score_body,
        grid=(_NPAD // _D,),
        in_specs=[
            pl.BlockSpec((_D, _D), lambda i: (i, 0)),
            pl.BlockSpec((_D, _D), lambda i: (i, 0)),
            pl.BlockSpec((_D, _D), lambda i: (i, 0)),
        ],
        out_specs=pl.BlockSpec((1, 1, _D), lambda i: (i, 0, 0)),
        out_shape=jax.ShapeDtypeStruct((_NPAD // _D, 1, _D), jnp.float32),
    )(feat_pad, mean_feat, max_feat)


def _key_of(s):
    b = lax.bitcast_convert_type(s, jnp.int32)
    return jnp.where(b >= 0, b, b ^ jnp.int32(0x7FFFFFFF))


def _topk_body(score_ref, feat_ref, ident_ref, su_ref, sl_ref,
               sample_ref, pscore_ref, pdesc_ref,
               scr_s, scr_gt, scr_eq, scr_acnt, scr_tcnt,
               row_sq, row_iq, row_sample, row_pscore):
    f32, i32 = jnp.float32, jnp.int32
    R = _NPAD // _D
    s_raw = score_ref[...]
    ridx = lax.broadcasted_iota(i32, (R, _D), 0)
    lidx = lax.broadcasted_iota(i32, (R, _D), 1)
    gidx = ridx * _D + lidx
    s = jnp.where(gidx < _N, s_raw, _NEG)
    s = jnp.where(s == 0.0, f32(0.0), s)
    key = _key_of(s)

    def bs(_, lohi):
        lo, hi = lohi
        mid = (lo >> 1) + (hi >> 1) + (lo & hi & 1)
        cnt = jnp.sum((key >= mid).astype(i32))
        big = cnt >= _KPT
        return (jnp.where(big, mid, lo), jnp.where(big, hi, mid))

    lo, hi = lax.fori_loop(0, 32, bs,
                           (jnp.int32(-2**31), jnp.int32(2**31 - 1)))
    vstar = lo
    gtf = (key >= hi).astype(f32)
    eqf = (key == vstar).astype(f32)
    m_f = jnp.sum(gtf)

    strict_u = su_ref[...]
    strict_l = sl_ref[...]
    ident = ident_ref[...]

    def ex_cumsum(v):
        within = jnp.dot(v, strict_u, precision=_HI)
        rtot = jnp.sum(v, axis=1, keepdims=True)
        rpref = jnp.dot(strict_l, rtot, precision=_HI)
        return within + rpref

    scr_s[...] = s
    scr_gt[...] = gtf
    scr_eq[...] = eqf
    scr_acnt[...] = ex_cumsum(gtf)
    scr_tcnt[...] = ex_cumsum(eqf)

    CH = 8
    qs3 = lax.broadcasted_iota(i32, (_KPT, CH, _D), 0).astype(f32)
    l3 = lax.broadcasted_iota(i32, (1, CH, _D), 2)
    r3 = lax.broadcasted_iota(i32, (1, CH, _D), 1)

    def _red(x):
        return jnp.sum(jnp.sum(x, axis=2), axis=1, keepdims=True)

    def ext_body(c, carry):
        aidx_a, ascr_a, tie_a = carry
        rsl = pl.ds(c * CH, CH)
        a3 = scr_acnt[rsl, :][None]
        t3 = scr_tcnt[rsl, :][None]
        g3 = scr_gt[rsl, :][None]
        e3 = scr_eq[rsl, :][None]
        s3 = scr_s[rsl, :][None]
        gi3 = ((c * CH + r3) * _D + l3).astype(f32)
        amask = (a3 == qs3).astype(f32) * g3
        aidx_a = aidx_a + _red(amask * gi3)
        ascr_a = ascr_a + _red(amask * s3)
        tmask = ((m_f + t3) == qs3).astype(f32) * e3
        tie_a = tie_a + _red(tmask * gi3)
        return aidx_a, ascr_a, tie_a

    zc = jnp.zeros((_KPT, 1), f32)
    aidx_acc, ascr_acc, tie_acc = lax.fori_loop(0, R // CH, ext_body,
                                                (zc, zc, zc))

    qcol = lax.broadcasted_iota(i32, (_KPT, 1), 0).astype(f32)
    valid = qcol < m_f
    ascr = jnp.where(valid, ascr_acc, _NEG)

    def _t_col2row(x):
        return lax.dot_general(x, ident, (((0,), (0,)), ((), ())),
                               precision=_HI)

    def _t_row2col(x):
        return lax.dot_general(ident, x, (((1,), (1,)), ((), ())),
                               precision=_HI)

    row_sq[...] = _t_col2row(ascr)
    row_iq[...] = _t_col2row(aidx_acc)

    QC = 128

    def rank_body(cc, racc):
        sq = row_sq[:, pl.ds(cc * QC, QC)]
        iq = row_iq[:, pl.ds(cc * QC, QC)]
        before = (sq > ascr) | ((sq == ascr) & (iq < aidx_acc))
        return racc + jnp.sum(before.astype(f32), axis=1, keepdims=True)

    rank = lax.fori_loop(0, _KPT // QC, rank_body, zc)

    def scat_body(cc, _):
        qrow = (cc * QC
                + lax.broadcasted_iota(i32, (1, QC), 1)).astype(f32)
        smat = ((rank == qrow) & valid).astype(f32)
        row_sample[:, pl.ds(cc * QC, QC)] = jnp.sum(
            smat * aidx_acc, axis=0, keepdims=True)
        row_pscore[:, pl.ds(cc * QC, QC)] = jnp.sum(
            smat * ascr_acc, axis=0, keepdims=True)
        return 0

    lax.fori_loop(0, _KPT // QC, scat_body, 0)

    qrow_full = lax.broadcasted_iota(i32, (1, _KPT), 1).astype(f32)
    sample_row = row_sample[...] + _t_col2row(tie_acc)
    vbits = jnp.where(vstar >= 0, vstar, vstar ^ i32(0x7FFFFFFF))
    tie_score = lax.bitcast_convert_type(vbits, f32)
    pred_score = jnp.where(qrow_full < m_f, row_pscore[...], tie_score)

    sample_col = _t_row2col(sample_row)
    FC = 1024

    def desc_body(c, dacc):
        fchunk = feat_ref[pl.ds(c * FC, FC), :]
        colid = (c * FC
                 + lax.broadcasted_iota(i32, (1, FC), 1)).astype(f32)
        oh = (sample_col == colid).astype(f32)
        return dacc + jnp.dot(oh, fchunk, precision=_HI)

    acc = lax.fori_loop(0, _NPAD // FC, desc_body,
                        jnp.zeros((_KPT, _D), f32))
    nrm = jnp.sqrt(jnp.sum(acc * acc, axis=1, keepdims=True)) + 1e-12
    pdesc_ref[...] = acc / nrm
    sample_ref[...] = sample_row.astype(i32)
    pscore_ref[...] = pred_score


def _tc_topk(score_tile, feat_pad):
    ident = jnp.eye(_KPT, dtype=jnp.float32)
    strict_u = jnp.triu(jnp.ones((_D, _D), jnp.float32), k=1)
    strict_l = jnp.tril(jnp.ones((_NPAD // _D, _NPAD // _D), jnp.float32),
                        k=-1)
    return pl.pallas_call(
        _topk_body,
        out_shape=[
            jax.ShapeDtypeStruct((1, _KPT), jnp.int32),
            jax.ShapeDtypeStruct((1, _KPT), jnp.float32),
            jax.ShapeDtypeStruct((_KPT, _D), jnp.float32),
        ],
        scratch_shapes=[pltpu.VMEM((_NPAD // _D, _D), jnp.float32)] * 5
        + [pltpu.VMEM((1, _KPT), jnp.float32)] * 4,
    )(score_tile, feat_pad, ident, strict_u, strict_l)


def kernel(features, neighbors):
    nb_pad = jnp.zeros((_NPAD, _K), jnp.int32).at[:_N].set(neighbors)
    nbflat = nb_pad.reshape(-1)
    feat_pad = jnp.zeros((_NPAD, _D), jnp.float32).at[:_N].set(features)
    mean_feat, max_feat = _sc_neighbor_stats(features, nbflat)
    score_tile = _tc_score(feat_pad, mean_feat, max_feat).reshape(
        _NPAD // _D, _D)
    sample2, pscore2, pdescs = _tc_topk(score_tile, feat_pad)
    score = score_tile.reshape(-1)[:_N]
    return score, pscore2.reshape(_KPT), pdescs, sample2.reshape(_KPT)


# speedup vs baseline: 1.0041x; 1.0041x over previous
import functools

import jax
import jax.numpy as jnp
from jax import lax
from jax.experimental import pallas as pl
from jax.experimental.pallas import tpu as pltpu
from jax.experimental.pallas import tpu_sc as plsc

_N = 10000
_K = 32
_D = 128
_KPT = 512
_NPAD = 10240
_NW = 32
_PTS_W = _NPAD // _NW
_CPTS = 4
_NCH = _PTS_W // _CPTS
_NBUF = 4
_NEG = -3.0e38
_HI = jax.lax.Precision.HIGHEST


def _sc_neighbor_stats(features, ind, nbflat):
    mesh = plsc.VectorSubcoreMesh(core_axis_name="c", subcore_axis_name="s")
    ptsw = _NPAD // 16
    nch = ptsw // _CPTS
    crows = _CPTS * _K

    @functools.partial(
        pl.kernel,
        mesh=mesh,
        compiler_params=pltpu.CompilerParams(needs_layout_passes=False),
        out_type=[
            jax.ShapeDtypeStruct((_NPAD, _D), jnp.float32),
            jax.ShapeDtypeStruct((_NPAD, _D), jnp.float32),
        ],
        scratch_types=[
            pltpu.VMEM((ptsw * _K,), jnp.int32),
            pltpu.VMEM((_NPAD,), jnp.float32),
            pltpu.VMEM((ptsw,), jnp.float32),
            pltpu.VMEM((_NBUF, crows, _D), jnp.float32),
            pltpu.VMEM((_CPTS, _D), jnp.float32),
            pltpu.VMEM((_CPTS, _D), jnp.float32),
            pltpu.SemaphoreType.DMA,
            pltpu.SemaphoreType.DMA,
            pltpu.SemaphoreType.DMA,
            pltpu.SemaphoreType.DMA,
        ],
    )
    def k(feat_hbm, ind_hbm, nb_hbm, mean_hbm, max_hbm,
          nb_v, ind_t, cnt_v, rows_v, mean_v, max_v, s0, s1, s2, s3):
        sems = (s0, s1, s2, s3)
        cid = lax.axis_index("c")
        sid = lax.axis_index("s")

        @pl.when(cid == 0)
        def _work():
            start = sid * ptsw
            pltpu.sync_copy(nb_hbm.at[pl.ds(start * _K, ptsw * _K)], nb_v)
            pltpu.sync_copy(ind_hbm, ind_t)
            iota16 = lax.broadcasted_iota(jnp.int32, (16,), 0)

            def cnt_body(pg, carry):
                cnt16 = jnp.zeros((16,), jnp.float32)
                for kk in range(_K):
                    offs = (pg * 16 + iota16) * _K + kk
                    nbv = plsc.load_gather(nb_v, [offs])
                    cnt16 = cnt16 + plsc.load_gather(ind_t, [nbv])
                cnt_v[pl.ds(pg * 16, 16)] = jnp.maximum(cnt16, 1.0)
                return carry

            lax.fori_loop(0, ptsw // 16, cnt_body, jnp.int32(0))

            def fire(c, b):
                idx_sl = nb_v.at[pl.ds(c * crows, crows)]
                pltpu.async_copy(feat_hbm.at[idx_sl], rows_v.at[b], sems[b])

            def wait(c, b):
                idx_sl = nb_v.at[pl.ds(c * crows, crows)]
                pltpu.make_async_copy(feat_hbm.at[idx_sl], rows_v.at[b],
                                      sems[b]).wait()

            def compute_chunk(c, b):
                base = start + c * _CPTS

                def pt_body(p, carry):
                    accs = [None] * 8
                    accm = [None] * 8
                    for r in range(_K):
                        vs = [rows_v[b, p * _K + r, pl.ds(j * 16, 16)]
                              for j in range(8)]
                        if r == 0:
                            accs = list(vs)
                            accm = list(vs)
                        else:
                            accs = [a + v for a, v in zip(accs, vs)]
                            accm = [jnp.maximum(a, v)
                                    for a, v in zip(accm, vs)]
                    csplat = jnp.full((16,), c * _CPTS + p, jnp.int32)
                    cs = plsc.load_gather(cnt_v, [csplat])
                    for j in range(8):
                        mean_v[p, pl.ds(j * 16, 16)] = accs[j] / cs
                        max_v[p, pl.ds(j * 16, 16)] = accm[j]
                    return carry

                lax.fori_loop(0, _CPTS, pt_body, jnp.int32(0))
                pltpu.sync_copy(mean_v, mean_hbm.at[pl.ds(base, _CPTS)])
                pltpu.sync_copy(max_v, max_hbm.at[pl.ds(base, _CPTS)])

            for b in range(_NBUF):
                fire(jnp.int32(b), b)

            def group_body(g, carry):
                for b in range(_NBUF):
                    c = g * _NBUF + b
                    wait(c, b)
                    compute_chunk(c, b)
                    fire(lax.rem(c + _NBUF, nch), b)
                return carry

            lax.fori_loop(0, nch // _NBUF, group_body, jnp.int32(0))
            for b in range(_NBUF):
                wait(jnp.int32(b), b)

    return k(features, ind, nbflat)


def _ind_body(x_ref, out_ref):
    rs = jnp.sum(x_ref[...], axis=1)
    out_ref[...] = (rs != 0.0).astype(jnp.float32)[None, None, :]


def _tc_ind(feat_pad):
    out = pl.pallas_call(
        _ind_body,
        grid=(_NPAD // 1024,),
        in_specs=[pl.BlockSpec((1024, _D), lambda i: (i, 0))],
        out_specs=pl.BlockSpec((1, 1, 1024), lambda i: (i, 0, 0)),
        out_shape=jax.ShapeDtypeStruct((_NPAD // 1024, 1, 1024), jnp.float32),
    )(feat_pad)
    return out.reshape(-1)


def _score_body(x_ref, mean_ref, mx_ref, out_ref):
    x = x_ref[...]
    mean = mean_ref[...]
    mx = mx_ref[...]
    rowmax = jnp.max(x, axis=1, keepdims=True)
    beta = x / (rowmax + 1e-6)
    alpha = jax.nn.softplus(x - mean)
    score_map = jnp.max(alpha * beta, axis=1)
    detected = jnp.max((x == mx).astype(jnp.float32), axis=1)
    out_ref[...] = (score_map * detected)[None, None, :]


def _tc_score(feat_pad, mean_feat, max_feat):
    return pl.pallas_call(
        _score_body,
        grid=(_NPAD // _D,),
        in_specs=[
            pl.BlockSpec((_D, _D), lambda i: (i, 0)),
            pl.BlockSpec((_D, _D), lambda i: (i, 0)),
            pl.BlockSpec((_D, _D), lambda i: (i, 0)),
        ],
        out_specs=pl.BlockSpec((1, 1, _D), lambda i: (i, 0, 0)),
        out_shape=jax.ShapeDtypeStruct((_NPAD // _D, 1, _D), jnp.float32),
    )(feat_pad, mean_feat, max_feat)


def _key_of(s):
    b = lax.bitcast_convert_type(s, jnp.int32)
    return jnp.where(b >= 0, b, b ^ jnp.int32(0x7FFFFFFF))


def _topk_body(score_ref, feat_ref, ident_ref, su_ref, sl_ref,
               sample_ref, pscore_ref, pdesc_ref,
               scr_s, scr_gt, scr_eq, scr_acnt, scr_tcnt,
               row_sq, row_iq, row_sample, row_pscore):
    f32, i32 = jnp.float32, jnp.int32
    R = _NPAD // _D
    s_raw = score_ref[...]
    ridx = lax.broadcasted_iota(i32, (R, _D), 0)
    lidx = lax.broadcasted_iota(i32, (R, _D), 1)
    gidx = ridx * _D + lidx
    s = jnp.where(gidx < _N, s_raw, _NEG)
    s = jnp.where(s == 0.0, f32(0.0), s)
    key = _key_of(s)

    def bs(_, lohi):
        lo, hi = lohi
        mid = (lo >> 1) + (hi >> 1) + (lo & hi & 1)
        cnt = jnp.sum((key >= mid).astype(i32))
        big = cnt >= _KPT
        return (jnp.where(big, mid, lo), jnp.where(big, hi, mid))

    lo, hi = lax.fori_loop(0, 32, bs,
                           (jnp.int32(-2**31), jnp.int32(2**31 - 1)))
    vstar = lo
    gtf = (key >= hi).astype(f32)
    eqf = (key == vstar).astype(f32)
    m_f = jnp.sum(gtf)

    strict_u = su_ref[...]
    strict_l = sl_ref[...]
    ident = ident_ref[...]

    def ex_cumsum(v):
        within = jnp.dot(v, strict_u, precision=_HI)
        rtot = jnp.sum(v, axis=1, keepdims=True)
        rpref = jnp.dot(strict_l, rtot, precision=_HI)
        return within + rpref

    scr_s[...] = s
    scr_gt[...] = gtf
    scr_eq[...] = eqf
    scr_acnt[...] = ex_cumsum(gtf)
    scr_tcnt[...] = ex_cumsum(eqf)

    CH = 8
    qs3 = lax.broadcasted_iota(i32, (_KPT, CH, _D), 0).astype(f32)
    l3 = lax.broadcasted_iota(i32, (1, CH, _D), 2)
    r3 = lax.broadcasted_iota(i32, (1, CH, _D), 1)

    def _red(x):
        return jnp.sum(jnp.sum(x, axis=2), axis=1, keepdims=True)

    def ext_body(c, carry):
        aidx_a, ascr_a, tie_a = carry
        rsl = pl.ds(c * CH, CH)
        a3 = scr_acnt[rsl, :][None]
        t3 = scr_tcnt[rsl, :][None]
        g3 = scr_gt[rsl, :][None]
        e3 = scr_eq[rsl, :][None]
        s3 = scr_s[rsl, :][None]
        gi3 = ((c * CH + r3) * _D + l3).astype(f32)
        amask = (a3 == qs3).astype(f32) * g3
        aidx_a = aidx_a + _red(amask * gi3)
        ascr_a = ascr_a + _red(amask * s3)
        tmask = ((m_f + t3) == qs3).astype(f32) * e3
        tie_a = tie_a + _red(tmask * gi3)
        return aidx_a, ascr_a, tie_a

    zc = jnp.zeros((_KPT, 1), f32)
    aidx_acc, ascr_acc, tie_acc = lax.fori_loop(0, R // CH, ext_body,
                                                (zc, zc, zc))

    qcol = lax.broadcasted_iota(i32, (_KPT, 1), 0).astype(f32)
    valid = qcol < m_f
    ascr = jnp.where(valid, ascr_acc, _NEG)

    def _t_col2row(x):
        return lax.dot_general(x, ident, (((0,), (0,)), ((), ())),
                               precision=_HI)

    def _t_row2col(x):
        return lax.dot_general(ident, x, (((1,), (1,)), ((), ())),
                               precision=_HI)

    row_sq[...] = _t_col2row(ascr)
    row_iq[...] = _t_col2row(aidx_acc)

    QC = 128

    def rank_body(cc, racc):
        sq = row_sq[:, pl.ds(cc * QC, QC)]
        iq = row_iq[:, pl.ds(cc * QC, QC)]
        before = (sq > ascr) | ((sq == ascr) & (iq < aidx_acc))
        return racc + jnp.sum(before.astype(f32), axis=1, keepdims=True)

    rank = lax.fori_loop(0, _KPT // QC, rank_body, zc)

    def scat_body(cc, _):
        qrow = (cc * QC
                + lax.broadcasted_iota(i32, (1, QC), 1)).astype(f32)
        smat = ((rank == qrow) & valid).astype(f32)
        row_sample[:, pl.ds(cc * QC, QC)] = jnp.sum(
            smat * aidx_acc, axis=0, keepdims=True)
        row_pscore[:, pl.ds(cc * QC, QC)] = jnp.sum(
            smat * ascr_acc, axis=0, keepdims=True)
        return 0

    lax.fori_loop(0, _KPT // QC, scat_body, 0)

    qrow_full = lax.broadcasted_iota(i32, (1, _KPT), 1).astype(f32)
    sample_row = row_sample[...] + _t_col2row(tie_acc)
    vbits = jnp.where(vstar >= 0, vstar, vstar ^ i32(0x7FFFFFFF))
    tie_score = lax.bitcast_convert_type(vbits, f32)
    pred_score = jnp.where(qrow_full < m_f, row_pscore[...], tie_score)

    sample_col = _t_row2col(sample_row)
    FC = 1024

    def desc_body(c, dacc):
        fchunk = feat_ref[pl.ds(c * FC, FC), :]
        colid = (c * FC
                 + lax.broadcasted_iota(i32, (1, FC), 1)).astype(f32)
        oh = (sample_col == colid).astype(f32)
        return dacc + jnp.dot(oh, fchunk, precision=_HI)

    acc = lax.fori_loop(0, _NPAD // FC, desc_body,
                        jnp.zeros((_KPT, _D), f32))
    nrm = jnp.sqrt(jnp.sum(acc * acc, axis=1, keepdims=True)) + 1e-12
    pdesc_ref[...] = acc / nrm
    sample_ref[...] = sample_row.astype(i32)
    pscore_ref[...] = pred_score


def _tc_topk(score_tile, feat_pad):
    ident = jnp.eye(_KPT, dtype=jnp.float32)
    strict_u = jnp.triu(jnp.ones((_D, _D), jnp.float32), k=1)
    strict_l = jnp.tril(jnp.ones((_NPAD // _D, _NPAD // _D), jnp.float32),
                        k=-1)
    return pl.pallas_call(
        _topk_body,
        out_shape=[
            jax.ShapeDtypeStruct((1, _KPT), jnp.int32),
            jax.ShapeDtypeStruct((1, _KPT), jnp.float32),
            jax.ShapeDtypeStruct((_KPT, _D), jnp.float32),
        ],
        scratch_shapes=[pltpu.VMEM((_NPAD // _D, _D), jnp.float32)] * 5
        + [pltpu.VMEM((1, _KPT), jnp.float32)] * 4,
    )(score_tile, feat_pad, ident, strict_u, strict_l)


def kernel(features, neighbors):
    nb_pad = jnp.zeros((_NPAD, _K), jnp.int32).at[:_N].set(neighbors)
    nbflat = nb_pad.reshape(-1)
    feat_pad = jnp.zeros((_NPAD, _D), jnp.float32).at[:_N].set(features)
    ind = _tc_ind(feat_pad)
    mean_feat, max_feat = _sc_neighbor_stats(features, ind, nbflat)
    score_tile = _tc_score(feat_pad, mean_feat, max_feat).reshape(
        _NPAD // _D, _D)
    sample2, pscore2, pdescs = _tc_topk(score_tile, feat_pad)
    score = score_tile.reshape(-1)[:_N]
    return score, pscore2.reshape(_KPT), pdescs, sample2.reshape(_KPT)


# speedup vs baseline: 1.1765x; 1.1717x over previous
import functools

import jax
import jax.numpy as jnp
from jax import lax
from jax.experimental import pallas as pl
from jax.experimental.pallas import tpu as pltpu
from jax.experimental.pallas import tpu_sc as plsc

_N = 10000
_K = 32
_D = 128
_KPT = 512
_NPAD = 10240
_NW = 32
_PTS_W = _NPAD // _NW
_CPTS = 4
_NCH = _PTS_W // _CPTS
_NBUF = 4
_PTS0 = 576
_PTS1 = 64
_NEG = -3.0e38
_HI = jax.lax.Precision.HIGHEST


def _sc_neighbor_stats(features, ind, nbflat):
    mesh = plsc.VectorSubcoreMesh(core_axis_name="c", subcore_axis_name="s")

    @functools.partial(
        pl.kernel,
        mesh=mesh,
        compiler_params=pltpu.CompilerParams(needs_layout_passes=False),
        out_type=[
            jax.ShapeDtypeStruct((_NPAD, _D), jnp.float32),
            jax.ShapeDtypeStruct((_NPAD, _D), jnp.float32),
        ],
        scratch_types=[
            pltpu.VMEM((_PTS0 * _K,), jnp.int32),
            pltpu.VMEM((_NPAD,), jnp.float32),
            pltpu.VMEM((_PTS0,), jnp.float32),
            pltpu.VMEM((_NBUF, _CPTS * _K, _D), jnp.float32),
            pltpu.VMEM((_CPTS, _D), jnp.float32),
            pltpu.VMEM((_CPTS, _D), jnp.float32),
            pltpu.SemaphoreType.DMA,
            pltpu.SemaphoreType.DMA,
            pltpu.SemaphoreType.DMA,
            pltpu.SemaphoreType.DMA,
        ],
    )
    def k(feat_hbm, ind_hbm, nb_hbm, mean_hbm, max_hbm,
          nb_v, ind_t, cnt_v, rows_v, mean_v, max_v, s0, s1, s2, s3):
        sems = (s0, s1, s2, s3)
        cid = lax.axis_index("c")
        sid = lax.axis_index("s")
        pts = jnp.where(cid == 0, _PTS0, _PTS1)
        start = jnp.where(cid == 0, sid * _PTS0, 16 * _PTS0 + sid * _PTS1)
        nch = pts // _CPTS
        pltpu.sync_copy(nb_hbm.at[pl.ds(start * _K, _PTS0 * _K)], nb_v)
        pltpu.sync_copy(ind_hbm, ind_t)
        iota16 = lax.broadcasted_iota(jnp.int32, (16,), 0)

        def cnt_body(pg, carry):
            cnt16 = jnp.zeros((16,), jnp.float32)
            for kk in range(_K):
                offs = (pg * 16 + iota16) * _K + kk
                nbv = plsc.load_gather(nb_v, [offs])
                cnt16 = cnt16 + plsc.load_gather(ind_t, [nbv])
            cnt_v[pl.ds(pg * 16, 16)] = jnp.maximum(cnt16, 1.0)
            return carry

        lax.fori_loop(0, pts // 16, cnt_body, jnp.int32(0))

        off16 = cid * _N + jnp.zeros((16,), jnp.int32)

        def off_body(i, carry):
            sl = pl.ds(i * 16, 16)
            nb_v[sl] = nb_v[sl] + off16
            return carry

        lax.fori_loop(0, pts * _K // 16, off_body, jnp.int32(0))

        crows = _CPTS * _K

        def fire(c, b):
            idx_sl = nb_v.at[pl.ds(c * crows, crows)]
            pltpu.async_copy(feat_hbm.at[idx_sl], rows_v.at[b], sems[b])

        def wait(c, b):
            idx_sl = nb_v.at[pl.ds(c * crows, crows)]
            pltpu.make_async_copy(feat_hbm.at[idx_sl], rows_v.at[b],
                                  sems[b]).wait()

        def compute_chunk(c, b):
            base = start + c * _CPTS

            def pt_body(p, carry):
                accs = [None] * 8
                accm = [None] * 8
                for r in range(_K):
                    vs = [rows_v[b, p * _K + r, pl.ds(j * 16, 16)]
                          for j in range(8)]
                    if r == 0:
                        accs = list(vs)
                        accm = list(vs)
                    else:
                        accs = [a + v for a, v in zip(accs, vs)]
                        accm = [jnp.maximum(a, v) for a, v in zip(accm, vs)]
                csplat = jnp.full((16,), c * _CPTS + p, jnp.int32)
                cs = plsc.load_gather(cnt_v, [csplat])
                for j in range(8):
                    mean_v[p, pl.ds(j * 16, 16)] = accs[j] / cs
                    max_v[p, pl.ds(j * 16, 16)] = accm[j]
                return carry

            lax.fori_loop(0, _CPTS, pt_body, jnp.int32(0))
            pltpu.sync_copy(mean_v, mean_hbm.at[pl.ds(base, _CPTS)])
            pltpu.sync_copy(max_v, max_hbm.at[pl.ds(base, _CPTS)])

        for b in range(_NBUF):
            fire(jnp.int32(b), b)

        def group_body(g, carry):
            for b in range(_NBUF):
                c = g * _NBUF + b
                wait(c, b)
                compute_chunk(c, b)
                fire(lax.rem(c + _NBUF, nch), b)
            return carry

        lax.fori_loop(0, nch // _NBUF, group_body, jnp.int32(0))
        for b in range(_NBUF):
            wait(jnp.int32(b), b)

    return k(features, ind, nbflat)


def _ind_body(x_ref, out_ref):
    rs = jnp.sum(x_ref[...], axis=1)
    out_ref[...] = (rs != 0.0).astype(jnp.float32)[None, None, :]


def _tc_ind(feat_pad):
    out = pl.pallas_call(
        _ind_body,
        grid=(_NPAD // 1024,),
        in_specs=[pl.BlockSpec((1024, _D), lambda i: (i, 0))],
        out_specs=pl.BlockSpec((1, 1, 1024), lambda i: (i, 0, 0)),
        out_shape=jax.ShapeDtypeStruct((_NPAD // 1024, 1, 1024), jnp.float32),
    )(feat_pad)
    return out.reshape(-1)


def _score_body(x_ref, mean_ref, mx_ref, out_ref):
    x = x_ref[...]
    mean = mean_ref[...]
    mx = mx_ref[...]
    rowmax = jnp.max(x, axis=1, keepdims=True)
    beta = x / (rowmax + 1e-6)
    alpha = jax.nn.softplus(x - mean)
    score_map = jnp.max(alpha * beta, axis=1)
    detected = jnp.max((x == mx).astype(jnp.float32), axis=1)
    out_ref[...] = (score_map * detected)[None, None, :]


def _tc_score(feat_pad, mean_feat, max_feat):
    return pl.pallas_call(
        _score_body,
        grid=(_NPAD // _D,),
        in_specs=[
            pl.BlockSpec((_D, _D), lambda i: (i, 0)),
            pl.BlockSpec((_D, _D), lambda i: (i, 0)),
            pl.BlockSpec((_D, _D), lambda i: (i, 0)),
        ],
        out_specs=pl.BlockSpec((1, 1, _D), lambda i: (i, 0, 0)),
        out_shape=jax.ShapeDtypeStruct((_NPAD // _D, 1, _D), jnp.float32),
    )(feat_pad, mean_feat, max_feat)


def _key_of(s):
    b = lax.bitcast_convert_type(s, jnp.int32)
    return jnp.where(b >= 0, b, b ^ jnp.int32(0x7FFFFFFF))


def _topk_body(score_ref, feat_ref, ident_ref, su_ref, sl_ref,
               sample_ref, pscore_ref, pdesc_ref,
               scr_s, scr_gt, scr_eq, scr_acnt, scr_tcnt,
               row_sq, row_iq, row_sample, row_pscore):
    f32, i32 = jnp.float32, jnp.int32
    R = _NPAD // _D
    s_raw = score_ref[...]
    ridx = lax.broadcasted_iota(i32, (R, _D), 0)
    lidx = lax.broadcasted_iota(i32, (R, _D), 1)
    gidx = ridx * _D + lidx
    s = jnp.where(gidx < _N, s_raw, _NEG)
    s = jnp.where(s == 0.0, f32(0.0), s)
    key = _key_of(s)

    def bs(_, lohi):
        lo, hi = lohi
        mid = (lo >> 1) + (hi >> 1) + (lo & hi & 1)
        cnt = jnp.sum((key >= mid).astype(i32))
        big = cnt >= _KPT
        return (jnp.where(big, mid, lo), jnp.where(big, hi, mid))

    lo, hi = lax.fori_loop(0, 32, bs,
                           (jnp.int32(-2**31), jnp.int32(2**31 - 1)))
    vstar = lo
    gtf = (key >= hi).astype(f32)
    eqf = (key == vstar).astype(f32)
    m_f = jnp.sum(gtf)

    strict_u = su_ref[...]
    strict_l = sl_ref[...]
    ident = ident_ref[...]

    def ex_cumsum(v):
        within = jnp.dot(v, strict_u, precision=_HI)
        rtot = jnp.sum(v, axis=1, keepdims=True)
        rpref = jnp.dot(strict_l, rtot, precision=_HI)
        return within + rpref

    scr_s[...] = s
    scr_gt[...] = gtf
    scr_eq[...] = eqf
    scr_acnt[...] = ex_cumsum(gtf)
    scr_tcnt[...] = ex_cumsum(eqf)

    CH = 8
    qs3 = lax.broadcasted_iota(i32, (_KPT, CH, _D), 0).astype(f32)
    l3 = lax.broadcasted_iota(i32, (1, CH, _D), 2)
    r3 = lax.broadcasted_iota(i32, (1, CH, _D), 1)

    def _red(x):
        return jnp.sum(jnp.sum(x, axis=2), axis=1, keepdims=True)

    def ext_body(c, carry):
        aidx_a, ascr_a, tie_a = carry
        rsl = pl.ds(c * CH, CH)
        a3 = scr_acnt[rsl, :][None]
        t3 = scr_tcnt[rsl, :][None]
        g3 = scr_gt[rsl, :][None]
        e3 = scr_eq[rsl, :][None]
        s3 = scr_s[rsl, :][None]
        gi3 = ((c * CH + r3) * _D + l3).astype(f32)
        amask = (a3 == qs3).astype(f32) * g3
        aidx_a = aidx_a + _red(amask * gi3)
        ascr_a = ascr_a + _red(amask * s3)
        tmask = ((m_f + t3) == qs3).astype(f32) * e3
        tie_a = tie_a + _red(tmask * gi3)
        return aidx_a, ascr_a, tie_a

    zc = jnp.zeros((_KPT, 1), f32)
    aidx_acc, ascr_acc, tie_acc = lax.fori_loop(0, R // CH, ext_body,
                                                (zc, zc, zc))

    qcol = lax.broadcasted_iota(i32, (_KPT, 1), 0).astype(f32)
    valid = qcol < m_f
    ascr = jnp.where(valid, ascr_acc, _NEG)

    def _t_col2row(x):
        return lax.dot_general(x, ident, (((0,), (0,)), ((), ())),
                               precision=_HI)

    def _t_row2col(x):
        return lax.dot_general(ident, x, (((1,), (1,)), ((), ())),
                               precision=_HI)

    row_sq[...] = _t_col2row(ascr)
    row_iq[...] = _t_col2row(aidx_acc)

    QC = 128

    def rank_body(cc, racc):
        sq = row_sq[:, pl.ds(cc * QC, QC)]
        iq = row_iq[:, pl.ds(cc * QC, QC)]
        before = (sq > ascr) | ((sq == ascr) & (iq < aidx_acc))
        return racc + jnp.sum(before.astype(f32), axis=1, keepdims=True)

    rank = lax.fori_loop(0, _KPT // QC, rank_body, zc)

    def scat_body(cc, _):
        qrow = (cc * QC
                + lax.broadcasted_iota(i32, (1, QC), 1)).astype(f32)
        smat = ((rank == qrow) & valid).astype(f32)
        row_sample[:, pl.ds(cc * QC, QC)] = jnp.sum(
            smat * aidx_acc, axis=0, keepdims=True)
        row_pscore[:, pl.ds(cc * QC, QC)] = jnp.sum(
            smat * ascr_acc, axis=0, keepdims=True)
        return 0

    lax.fori_loop(0, _KPT // QC, scat_body, 0)

    qrow_full = lax.broadcasted_iota(i32, (1, _KPT), 1).astype(f32)
    sample_row = row_sample[...] + _t_col2row(tie_acc)
    vbits = jnp.where(vstar >= 0, vstar, vstar ^ i32(0x7FFFFFFF))
    tie_score = lax.bitcast_convert_type(vbits, f32)
    pred_score = jnp.where(qrow_full < m_f, row_pscore[...], tie_score)

    sample_col = _t_row2col(sample_row)
    FC = 1024

    def desc_body(c, dacc):
        fchunk = feat_ref[pl.ds(c * FC, FC), :]
        colid = (c * FC
                 + lax.broadcasted_iota(i32, (1, FC), 1)).astype(f32)
        oh = (sample_col == colid).astype(f32)
        return dacc + jnp.dot(oh, fchunk, precision=_HI)

    acc = lax.fori_loop(0, _NPAD // FC, desc_body,
                        jnp.zeros((_KPT, _D), f32))
    nrm = jnp.sqrt(jnp.sum(acc * acc, axis=1, keepdims=True)) + 1e-12
    pdesc_ref[...] = acc / nrm
    sample_ref[...] = sample_row.astype(i32)
    pscore_ref[...] = pred_score


def _tc_topk(score_tile, feat_pad):
    ident = jnp.eye(_KPT, dtype=jnp.float32)
    strict_u = jnp.triu(jnp.ones((_D, _D), jnp.float32), k=1)
    strict_l = jnp.tril(jnp.ones((_NPAD // _D, _NPAD // _D), jnp.float32),
                        k=-1)
    return pl.pallas_call(
        _topk_body,
        out_shape=[
            jax.ShapeDtypeStruct((1, _KPT), jnp.int32),
            jax.ShapeDtypeStruct((1, _KPT), jnp.float32),
            jax.ShapeDtypeStruct((_KPT, _D), jnp.float32),
        ],
        scratch_shapes=[pltpu.VMEM((_NPAD // _D, _D), jnp.float32)] * 5
        + [pltpu.VMEM((1, _KPT), jnp.float32)] * 4,
    )(score_tile, feat_pad, ident, strict_u, strict_l)


def kernel(features, neighbors):
    nb_pad = jnp.zeros((_NPAD + _PTS0 - _PTS1, _K),
                       jnp.int32).at[:_N].set(neighbors)
    nbflat = nb_pad.reshape(-1)
    feat_pad = jnp.zeros((_NPAD, _D), jnp.float32).at[:_N].set(features)
    ind = _tc_ind(feat_pad)
    feat2 = jnp.concatenate([features, features], axis=0)
    mean_feat, max_feat = _sc_neighbor_stats(feat2, ind, nbflat)
    score_tile = _tc_score(feat_pad, mean_feat, max_feat).reshape(
        _NPAD // _D, _D)
    sample2, pscore2, pdescs = _tc_topk(score_tile, feat_pad)
    score = score_tile.reshape(-1)[:_N]
    return score, pscore2.reshape(_KPT), pdescs, sample2.reshape(_KPT)


# speedup vs baseline: 1.2958x; 1.1014x over previous
import functools

import jax
import jax.numpy as jnp
from jax import lax
from jax.experimental import pallas as pl
from jax.experimental.pallas import tpu as pltpu
from jax.experimental.pallas import tpu_sc as plsc

_N = 10000
_K = 32
_D = 128
_KPT = 512
_NPAD = 10240
_NW = 32
_PTS_W = _NPAD // _NW
_CPTS = 4
_NCH = _PTS_W // _CPTS
_NBUF = 4
_PTS0 = 320
_PTS1 = 320
_NEG = -3.0e38
_HI = jax.lax.Precision.HIGHEST


def _sc_neighbor_stats(features, ind, nbflat):
    mesh = plsc.VectorSubcoreMesh(core_axis_name="c", subcore_axis_name="s")

    @functools.partial(
        pl.kernel,
        mesh=mesh,
        compiler_params=pltpu.CompilerParams(needs_layout_passes=False),
        out_type=[
            jax.ShapeDtypeStruct((_NPAD, _D), jnp.float32),
            jax.ShapeDtypeStruct((_NPAD, _D), jnp.float32),
        ],
        scratch_types=[
            pltpu.VMEM((_PTS0 * _K,), jnp.int32),
            pltpu.VMEM((_NPAD,), jnp.float32),
            pltpu.VMEM((_PTS0,), jnp.float32),
            pltpu.VMEM((_NBUF, _CPTS * _K, _D), jnp.float32),
            pltpu.VMEM((_CPTS, _D), jnp.float32),
            pltpu.VMEM((_CPTS, _D), jnp.float32),
            pltpu.SemaphoreType.DMA,
            pltpu.SemaphoreType.DMA,
            pltpu.SemaphoreType.DMA,
            pltpu.SemaphoreType.DMA,
        ],
    )
    def k(feat_hbm, ind_hbm, nb_hbm, mean_hbm, max_hbm,
          nb_v, ind_t, cnt_v, rows_v, mean_v, max_v, s0, s1, s2, s3):
        sems = (s0, s1, s2, s3)
        cid = lax.axis_index("c")
        sid = lax.axis_index("s")
        pts = jnp.where(cid == 0, _PTS0, _PTS1)
        start = jnp.where(cid == 0, sid * _PTS0, 16 * _PTS0 + sid * _PTS1)
        nch = pts // _CPTS
        pltpu.sync_copy(nb_hbm.at[pl.ds(start * _K, _PTS0 * _K)], nb_v)
        pltpu.sync_copy(ind_hbm, ind_t)
        iota16 = lax.broadcasted_iota(jnp.int32, (16,), 0)

        def cnt_body(pg, carry):
            cnt16 = jnp.zeros((16,), jnp.float32)
            for kk in range(_K):
                offs = (pg * 16 + iota16) * _K + kk
                nbv = plsc.load_gather(nb_v, [offs])
                cnt16 = cnt16 + plsc.load_gather(ind_t, [nbv])
            cnt_v[pl.ds(pg * 16, 16)] = jnp.maximum(cnt16, 1.0)
            return carry

        lax.fori_loop(0, pts // 16, cnt_body, jnp.int32(0))

        off16 = cid * _N + jnp.zeros((16,), jnp.int32)

        def off_body(i, carry):
            sl = pl.ds(i * 16, 16)
            nb_v[sl] = nb_v[sl] + off16
            return carry

        lax.fori_loop(0, pts * _K // 16, off_body, jnp.int32(0))

        crows = _CPTS * _K

        def fire(c, b):
            idx_sl = nb_v.at[pl.ds(c * crows, crows)]
            pltpu.async_copy(feat_hbm.at[idx_sl], rows_v.at[b], sems[b])

        def wait(c, b):
            idx_sl = nb_v.at[pl.ds(c * crows, crows)]
            pltpu.make_async_copy(feat_hbm.at[idx_sl], rows_v.at[b],
                                  sems[b]).wait()

        def compute_chunk(c, b):
            base = start + c * _CPTS

            def pt_body(p, carry):
                accs = [None] * 8
                accm = [None] * 8
                for r in range(_K):
                    vs = [rows_v[b, p * _K + r, pl.ds(j * 16, 16)]
                          for j in range(8)]
                    if r == 0:
                        accs = list(vs)
                        accm = list(vs)
                    else:
                        accs = [a + v for a, v in zip(accs, vs)]
                        accm = [jnp.maximum(a, v) for a, v in zip(accm, vs)]
                csplat = jnp.full((16,), c * _CPTS + p, jnp.int32)
                cs = plsc.load_gather(cnt_v, [csplat])
                for j in range(8):
                    mean_v[p, pl.ds(j * 16, 16)] = accs[j] / cs
                    max_v[p, pl.ds(j * 16, 16)] = accm[j]
                return carry

            lax.fori_loop(0, _CPTS, pt_body, jnp.int32(0))
            pltpu.sync_copy(mean_v, mean_hbm.at[pl.ds(base, _CPTS)])
            pltpu.sync_copy(max_v, max_hbm.at[pl.ds(base, _CPTS)])

        for b in range(_NBUF):
            fire(jnp.int32(b), b)

        def group_body(g, carry):
            for b in range(_NBUF):
                c = g * _NBUF + b
                wait(c, b)
                compute_chunk(c, b)
                fire(lax.rem(c + _NBUF, nch), b)
            return carry

        lax.fori_loop(0, nch // _NBUF, group_body, jnp.int32(0))
        for b in range(_NBUF):
            wait(jnp.int32(b), b)

    return k(features, ind, nbflat)


def _ind_body(x_ref, out_ref):
    rs = jnp.sum(x_ref[...], axis=1)
    out_ref[...] = (rs != 0.0).astype(jnp.float32)[None, None, :]


def _tc_ind(feat_pad):
    out = pl.pallas_call(
        _ind_body,
        grid=(_NPAD // 1024,),
        in_specs=[pl.BlockSpec((1024, _D), lambda i: (i, 0))],
        out_specs=pl.BlockSpec((1, 1, 1024), lambda i: (i, 0, 0)),
        out_shape=jax.ShapeDtypeStruct((_NPAD // 1024, 1, 1024), jnp.float32),
    )(feat_pad)
    return out.reshape(-1)


def _score_body(x_ref, mean_ref, mx_ref, out_ref):
    x = x_ref[...]
    mean = mean_ref[...]
    mx = mx_ref[...]
    rowmax = jnp.max(x, axis=1, keepdims=True)
    beta = x / (rowmax + 1e-6)
    alpha = jax.nn.softplus(x - mean)
    score_map = jnp.max(alpha * beta, axis=1)
    detected = jnp.max((x == mx).astype(jnp.float32), axis=1)
    out_ref[...] = (score_map * detected)[None, None, :]


def _tc_score(feat_pad, mean_feat, max_feat):
    return pl.pallas_call(
        _score_body,
        grid=(_NPAD // _D,),
        in_specs=[
            pl.BlockSpec((_D, _D), lambda i: (i, 0)),
            pl.BlockSpec((_D, _D), lambda i: (i, 0)),
            pl.BlockSpec((_D, _D), lambda i: (i, 0)),
        ],
        out_specs=pl.BlockSpec((1, 1, _D), lambda i: (i, 0, 0)),
        out_shape=jax.ShapeDtypeStruct((_NPAD // _D, 1, _D), jnp.float32),
    )(feat_pad, mean_feat, max_feat)


def _key_of(s):
    b = lax.bitcast_convert_type(s, jnp.int32)
    return jnp.where(b >= 0, b, b ^ jnp.int32(0x7FFFFFFF))


def _topk_body(score_ref, feat_ref, ident_ref, su_ref, sl_ref,
               sample_ref, pscore_ref, pdesc_ref,
               scr_s, scr_gt, scr_eq, scr_acnt, scr_tcnt,
               row_sq, row_iq, row_sample, row_pscore):
    f32, i32 = jnp.float32, jnp.int32
    R = _NPAD // _D
    s_raw = score_ref[...]
    ridx = lax.broadcasted_iota(i32, (R, _D), 0)
    lidx = lax.broadcasted_iota(i32, (R, _D), 1)
    gidx = ridx * _D + lidx
    s = jnp.where(gidx < _N, s_raw, _NEG)
    s = jnp.where(s == 0.0, f32(0.0), s)
    key = _key_of(s)

    def bs(_, lohi):
        lo, hi = lohi
        mid = (lo >> 1) + (hi >> 1) + (lo & hi & 1)
        cnt = jnp.sum((key >= mid).astype(i32))
        big = cnt >= _KPT
        return (jnp.where(big, mid, lo), jnp.where(big, hi, mid))

    lo, hi = lax.fori_loop(0, 32, bs,
                           (jnp.int32(-2**31), jnp.int32(2**31 - 1)))
    vstar = lo
    gtf = (key >= hi).astype(f32)
    eqf = (key == vstar).astype(f32)
    m_f = jnp.sum(gtf)

    strict_u = su_ref[...]
    strict_l = sl_ref[...]
    ident = ident_ref[...]

    def ex_cumsum(v):
        within = jnp.dot(v, strict_u, precision=_HI)
        rtot = jnp.sum(v, axis=1, keepdims=True)
        rpref = jnp.dot(strict_l, rtot, precision=_HI)
        return within + rpref

    scr_s[...] = s
    scr_gt[...] = gtf
    scr_eq[...] = eqf
    scr_acnt[...] = ex_cumsum(gtf)
    scr_tcnt[...] = ex_cumsum(eqf)

    CH = 8
    qs3 = lax.broadcasted_iota(i32, (_KPT, CH, _D), 0).astype(f32)
    l3 = lax.broadcasted_iota(i32, (1, CH, _D), 2)
    r3 = lax.broadcasted_iota(i32, (1, CH, _D), 1)

    def _red(x):
        return jnp.sum(jnp.sum(x, axis=2), axis=1, keepdims=True)

    def ext_body(c, carry):
        aidx_a, ascr_a, tie_a = carry
        rsl = pl.ds(c * CH, CH)
        a3 = scr_acnt[rsl, :][None]
        t3 = scr_tcnt[rsl, :][None]
        g3 = scr_gt[rsl, :][None]
        e3 = scr_eq[rsl, :][None]
        s3 = scr_s[rsl, :][None]
        gi3 = ((c * CH + r3) * _D + l3).astype(f32)
        amask = (a3 == qs3).astype(f32) * g3
        aidx_a = aidx_a + _red(amask * gi3)
        ascr_a = ascr_a + _red(amask * s3)
        tmask = ((m_f + t3) == qs3).astype(f32) * e3
        tie_a = tie_a + _red(tmask * gi3)
        return aidx_a, ascr_a, tie_a

    zc = jnp.zeros((_KPT, 1), f32)
    aidx_acc, ascr_acc, tie_acc = lax.fori_loop(0, R // CH, ext_body,
                                                (zc, zc, zc))

    qcol = lax.broadcasted_iota(i32, (_KPT, 1), 0).astype(f32)
    valid = qcol < m_f
    ascr = jnp.where(valid, ascr_acc, _NEG)

    def _t_col2row(x):
        return lax.dot_general(x, ident, (((0,), (0,)), ((), ())),
                               precision=_HI)

    def _t_row2col(x):
        return lax.dot_general(ident, x, (((1,), (1,)), ((), ())),
                               precision=_HI)

    row_sq[...] = _t_col2row(ascr)
    row_iq[...] = _t_col2row(aidx_acc)

    QC = 128

    def rank_body(cc, racc):
        sq = row_sq[:, pl.ds(cc * QC, QC)]
        iq = row_iq[:, pl.ds(cc * QC, QC)]
        before = (sq > ascr) | ((sq == ascr) & (iq < aidx_acc))
        return racc + jnp.sum(before.astype(f32), axis=1, keepdims=True)

    rank = lax.fori_loop(0, _KPT // QC, rank_body, zc)

    def scat_body(cc, _):
        qrow = (cc * QC
                + lax.broadcasted_iota(i32, (1, QC), 1)).astype(f32)
        smat = ((rank == qrow) & valid).astype(f32)
        row_sample[:, pl.ds(cc * QC, QC)] = jnp.sum(
            smat * aidx_acc, axis=0, keepdims=True)
        row_pscore[:, pl.ds(cc * QC, QC)] = jnp.sum(
            smat * ascr_acc, axis=0, keepdims=True)
        return 0

    lax.fori_loop(0, _KPT // QC, scat_body, 0)

    qrow_full = lax.broadcasted_iota(i32, (1, _KPT), 1).astype(f32)
    sample_row = row_sample[...] + _t_col2row(tie_acc)
    vbits = jnp.where(vstar >= 0, vstar, vstar ^ i32(0x7FFFFFFF))
    tie_score = lax.bitcast_convert_type(vbits, f32)
    pred_score = jnp.where(qrow_full < m_f, row_pscore[...], tie_score)

    sample_col = _t_row2col(sample_row)
    FC = 1024

    def desc_body(c, dacc):
        fchunk = feat_ref[pl.ds(c * FC, FC), :]
        colid = (c * FC
                 + lax.broadcasted_iota(i32, (1, FC), 1)).astype(f32)
        oh = (sample_col == colid).astype(f32)
        return dacc + jnp.dot(oh, fchunk, precision=_HI)

    acc = lax.fori_loop(0, _NPAD // FC, desc_body,
                        jnp.zeros((_KPT, _D), f32))
    nrm = jnp.sqrt(jnp.sum(acc * acc, axis=1, keepdims=True)) + 1e-12
    pdesc_ref[...] = acc / nrm
    sample_ref[...] = sample_row.astype(i32)
    pscore_ref[...] = pred_score


def _tc_topk(score_tile, feat_pad):
    ident = jnp.eye(_KPT, dtype=jnp.float32)
    strict_u = jnp.triu(jnp.ones((_D, _D), jnp.float32), k=1)
    strict_l = jnp.tril(jnp.ones((_NPAD // _D, _NPAD // _D), jnp.float32),
                        k=-1)
    return pl.pallas_call(
        _topk_body,
        out_shape=[
            jax.ShapeDtypeStruct((1, _KPT), jnp.int32),
            jax.ShapeDtypeStruct((1, _KPT), jnp.float32),
            jax.ShapeDtypeStruct((_KPT, _D), jnp.float32),
        ],
        scratch_shapes=[pltpu.VMEM((_NPAD // _D, _D), jnp.float32)] * 5
        + [pltpu.VMEM((1, _KPT), jnp.float32)] * 4,
    )(score_tile, feat_pad, ident, strict_u, strict_l)


def kernel(features, neighbors):
    nb_pad = jnp.zeros((_NPAD + _PTS0 - _PTS1, _K),
                       jnp.int32).at[:_N].set(neighbors)
    nbflat = nb_pad.reshape(-1)
    feat_pad = jnp.zeros((_NPAD, _D), jnp.float32).at[:_N].set(features)
    ind = _tc_ind(feat_pad)
    feat2 = jnp.concatenate([features, features], axis=0)
    mean_feat, max_feat = _sc_neighbor_stats(feat2, ind, nbflat)
    score_tile = _tc_score(feat_pad, mean_feat, max_feat).reshape(
        _NPAD // _D, _D)
    sample2, pscore2, pdescs = _tc_topk(score_tile, feat_pad)
    score = score_tile.reshape(-1)[:_N]
    return score, pscore2.reshape(_KPT), pdescs, sample2.reshape(_KPT)
